# Initial kernel scaffold; baseline (speedup 1.0000x reference)
import jax
import jax.numpy as jnp
from jax.experimental import pallas as pl


def kernel(token_ids, weight):
    raise NotImplementedError("write your pallas kernel here")


# speedup vs baseline: 4.9482x; 4.9482x over previous
import functools

import jax
import jax.numpy as jnp
from jax import lax
from jax.experimental import pallas as pl
from jax.experimental.pallas import tpu as pltpu
from jax.experimental.pallas import tpu_sc as plsc

D = 32

_info = plsc.get_sparse_core_info()
_NC, _NS = _info.num_cores, _info.num_subcores
NW = _NC * _NS

CHUNK = 2048


@functools.lru_cache(maxsize=None)
def _make_gather(B):
    assert B % (NW * CHUNK) == 0, (B, NW * CHUNK)
    n_per_w = B // NW
    n_chunks = n_per_w // CHUNK
    mesh = plsc.VectorSubcoreMesh(core_axis_name="c", subcore_axis_name="s")

    @functools.partial(
        pl.kernel,
        out_type=jax.ShapeDtypeStruct((B, D), jnp.float32),
        mesh=mesh,
        scratch_types=[
            pltpu.VMEM((CHUNK,), jnp.int32),
            pltpu.VMEM((CHUNK, D), jnp.float32),
            pltpu.SemaphoreType.DMA,
        ],
        compiler_params=pltpu.CompilerParams(use_tc_tiling_on_sc=False),
    )
    def gather_kernel(idx_hbm, table_hbm, out_hbm, idx_v, rows_v, sem):
        wid = lax.axis_index("s") * _NC + lax.axis_index("c")
        base = wid * n_per_w

        def body(i, carry):
            off = base + i * CHUNK
            pltpu.sync_copy(idx_hbm.at[pl.ds(off, CHUNK)], idx_v)
            pltpu.async_copy(table_hbm.at[idx_v], rows_v, sem).wait()
            pltpu.sync_copy(rows_v, out_hbm.at[pl.ds(off, CHUNK)])
            return carry

        lax.fori_loop(0, n_chunks, body, 0)

    return gather_kernel


@jax.jit
def kernel(token_ids, weight):
    b, h = token_ids.shape
    flat = token_ids.reshape(-1).astype(jnp.int32)
    out = _make_gather(b * h)(flat, weight)
    return out.reshape(b, h, weight.shape[1])


# speedup vs baseline: 4.9509x; 1.0005x over previous
import functools

import jax
import jax.numpy as jnp
from jax import lax
from jax.experimental import pallas as pl
from jax.experimental.pallas import tpu as pltpu
from jax.experimental.pallas import tpu_sc as plsc

D = 32

_info = plsc.get_sparse_core_info()
_NC, _NS = _info.num_cores, _info.num_subcores
NW = _NC * _NS

CHUNK = 800
NBUF = 4


@functools.lru_cache(maxsize=None)
def _make_gather(B):
    assert B % (NW * CHUNK) == 0, (B, NW * CHUNK)
    n_per_w = B // NW
    n_chunks = n_per_w // CHUNK
    assert n_chunks > NBUF
    mesh = plsc.VectorSubcoreMesh(core_axis_name="c", subcore_axis_name="s")

    @functools.partial(
        pl.kernel,
        out_type=jax.ShapeDtypeStruct((B, D), jnp.float32),
        mesh=mesh,
        scratch_types=[
            pltpu.VMEM((NBUF, CHUNK), jnp.int32),
            pltpu.VMEM((NBUF, CHUNK, D), jnp.float32),
            pltpu.SemaphoreType.DMA((NBUF,)),
            pltpu.SemaphoreType.DMA((NBUF,)),
        ],
        compiler_params=pltpu.CompilerParams(use_tc_tiling_on_sc=False),
    )
    def gather_kernel(idx_hbm, table_hbm, out_hbm, idx_v, rows_v, sem_g, sem_s):
        wid = lax.axis_index("s") * _NC + lax.axis_index("c")
        base = wid * n_per_w

        def start_gather(j, bj):
            off = base + j * CHUNK
            pltpu.sync_copy(idx_hbm.at[pl.ds(off, CHUNK)], idx_v.at[bj])
            pltpu.async_copy(table_hbm.at[idx_v.at[bj]], rows_v.at[bj],
                             sem_g.at[bj])

        for k in range(NBUF - 1):
            start_gather(k, k)

        def body(i, carry):
            b = lax.rem(i, NBUF)
            j = i + NBUF - 1
            bj = lax.rem(j, NBUF)

            @pl.when(j < n_chunks)
            def _():
                @pl.when(i > 0)
                def _():
                    pltpu.make_async_copy(
                        rows_v.at[bj], out_hbm.at[pl.ds(base, CHUNK)],
                        sem_s.at[bj]).wait()
                start_gather(j, bj)

            pltpu.make_async_copy(
                table_hbm.at[idx_v.at[b]], rows_v.at[b], sem_g.at[b]).wait()
            off = base + i * CHUNK
            pltpu.async_copy(rows_v.at[b], out_hbm.at[pl.ds(off, CHUNK)],
                             sem_s.at[b])
            return carry

        lax.fori_loop(0, n_chunks, body, 0)

        for bj in range(NBUF):
            pltpu.make_async_copy(
                rows_v.at[bj], out_hbm.at[pl.ds(base, CHUNK)],
                sem_s.at[bj]).wait()

    return gather_kernel


@jax.jit
def kernel(token_ids, weight):
    b, h = token_ids.shape
    flat = token_ids.reshape(-1).astype(jnp.int32)
    out = _make_gather(b * h)(flat, weight)
    return out.reshape(b, h, weight.shape[1])
